# baseline (speedup 1.0000x reference)
import jax
import jax.numpy as jnp
import numpy as np
from jax import lax
from jax.experimental import pallas as pl
from jax.experimental.pallas import tpu as pltpu
from jax.experimental.pallas import tpu_sc as plsc

NUM_ENT = 100000
NUM_REL = 32
DIM = 16
K = 16
B = 16384

NC = 2
NS = 16
NW = NC * NS
BPW = B // NW
H1_PER_W = BPW * K
H1_CHUNK = 1024
H2_CHUNK = 2048

_MESH = dict(core_axis_name="c", subcore_axis_name="s")
_SC_PARAMS = pltpu.CompilerParams(use_tc_tiling_on_sc=False,
                                  needs_layout_passes=False)


def _wid():
    return lax.axis_index("s") * NC + lax.axis_index("c")


def _sc1a_body(u_hbm, usr_hbm, ue_out, idx_v, ue_v, sem):
    sl = pl.ds(_wid() * BPW, BPW)
    pltpu.sync_copy(u_hbm.at[sl], idx_v)
    pltpu.async_copy(usr_hbm.at[idx_v], ue_v, sem).wait()
    pltpu.sync_copy(ue_v, ue_out.at[sl])


def _sc1a(u, usr_emb):
    run = pl.kernel(
        _sc1a_body,
        out_type=jax.ShapeDtypeStruct((B, DIM), jnp.float32),
        mesh=plsc.VectorSubcoreMesh(**_MESH),
        scratch_types=[
            pltpu.VMEM((BPW,), jnp.int32),
            pltpu.VMEM((BPW, DIM), jnp.float32),
            pltpu.SemaphoreType.DMA,
        ],
        compiler_params=_SC_PARAMS)
    return run(u, usr_emb)


def _sc1b_body(tn_hbm, ent_hbm, adje_hbm, adjr_hbm,
               v0_out, e1_out, r1_out,
               idx_v, v0_v, e1_v, r1_v, sem, sem2, sem3):
    sl = pl.ds(_wid() * BPW, BPW)
    pltpu.sync_copy(tn_hbm.at[sl], idx_v)
    g1 = pltpu.async_copy(ent_hbm.at[idx_v], v0_v, sem)
    g2 = pltpu.async_copy(adje_hbm.at[idx_v], e1_v, sem2)
    g3 = pltpu.async_copy(adjr_hbm.at[idx_v], r1_v, sem3)
    g1.wait()
    pltpu.sync_copy(v0_v, v0_out.at[sl])
    g2.wait()
    pltpu.sync_copy(e1_v, e1_out.at[sl])
    g3.wait()
    pltpu.sync_copy(r1_v, r1_out.at[sl])


def _sc1b(tn, ent_emb, adj_ent, adj_rel):
    f32, i32 = jnp.float32, jnp.int32
    out_type = (
        jax.ShapeDtypeStruct((B, DIM), f32),
        jax.ShapeDtypeStruct((B, K), i32),
        jax.ShapeDtypeStruct((B, K), i32),
    )
    scratch = [
        pltpu.VMEM((BPW,), i32),
        pltpu.VMEM((BPW, DIM), f32),
        pltpu.VMEM((BPW, K), i32),
        pltpu.VMEM((BPW, K), i32),
        pltpu.SemaphoreType.DMA,
        pltpu.SemaphoreType.DMA,
        pltpu.SemaphoreType.DMA,
    ]
    run = pl.kernel(_sc1b_body, out_type=out_type,
                    mesh=plsc.VectorSubcoreMesh(**_MESH),
                    scratch_types=scratch, compiler_params=_SC_PARAMS)
    return run(tn, ent_emb, adj_ent, adj_rel)


H1C = 512


def _sc2_body(e1_hbm, r1_hbm, es_hbm, ent_hbm, adje_hbm, adjr_hbm,
              v1_out, e2_out, u1_out, u2_out,
              es_v, r1_v, u1_v,
              idx0, idx1, v1a, v1b, e2a, e2b, r2a, r2b, u2a, u2b,
              s1a, s2a, s3a, s1b, s2b, s3b):
    w = _wid()
    b0 = w * BPW
    pltpu.sync_copy(es_hbm.at[pl.ds(b0, BPW)], es_v)
    pltpu.sync_copy(r1_hbm.at[pl.ds(b0, BPW)], r1_v)

    @plsc.parallel_loop(0, BPW, unroll=4)
    def u1_row(r):
        rows = jnp.full((16,), r, jnp.int32)
        u1_v[r] = plsc.load_gather(es_v, [rows, r1_v[r]])
    pltpu.sync_copy(u1_v, u1_out.at[pl.ds(b0, BPW)])

    def u2_loop(base_row, r2_v, u2_v):
        @plsc.parallel_loop(0, H1C, unroll=4)
        def u2_row(r):
            b_local = (base_row + r) // K
            rows = jnp.full((16,), b_local, jnp.int32)
            u2_v[r] = plsc.load_gather(es_v, [rows, r2_v[r]])

    def pair(i2, _):
        base0 = 2 * i2 * H1C
        base1 = base0 + H1C
        o0 = w * H1_PER_W + base0
        sl0 = pl.ds(o0, H1C)
        sl1 = pl.ds(o0 + H1C, H1C)
        pltpu.sync_copy(e1_hbm.at[sl0], idx0)
        g1a = pltpu.async_copy(ent_hbm.at[idx0], v1a, s1a)
        g2a = pltpu.async_copy(adje_hbm.at[idx0], e2a, s2a)
        g3a = pltpu.async_copy(adjr_hbm.at[idx0], r2a, s3a)
        pltpu.sync_copy(e1_hbm.at[sl1], idx1)
        g1b = pltpu.async_copy(ent_hbm.at[idx1], v1b, s1b)
        g2b = pltpu.async_copy(adje_hbm.at[idx1], e2b, s2b)
        g3b = pltpu.async_copy(adjr_hbm.at[idx1], r2b, s3b)
        g1a.wait()
        wa1 = pltpu.async_copy(v1a, v1_out.at[sl0], s1a)
        g2a.wait()
        wa2 = pltpu.async_copy(e2a, e2_out.at[sl0], s2a)
        g3a.wait()
        u2_loop(base0, r2a, u2a)
        wa3 = pltpu.async_copy(u2a, u2_out.at[sl0], s3a)
        g1b.wait()
        wb1 = pltpu.async_copy(v1b, v1_out.at[sl1], s1b)
        g2b.wait()
        wb2 = pltpu.async_copy(e2b, e2_out.at[sl1], s2b)
        g3b.wait()
        u2_loop(base1, r2b, u2b)
        wb3 = pltpu.async_copy(u2b, u2_out.at[sl1], s3b)
        wa1.wait()
        wa2.wait()
        wa3.wait()
        wb1.wait()
        wb2.wait()
        wb3.wait()
        return 0

    lax.fori_loop(0, H1_PER_W // (2 * H1C), pair, 0)


def _sc2(e1_flat, r1, es, ent_emb, adj_ent, adj_rel):
    f32, i32 = jnp.float32, jnp.int32
    out_type = (
        jax.ShapeDtypeStruct((B * K, DIM), f32),
        jax.ShapeDtypeStruct((B * K, K), i32),
        jax.ShapeDtypeStruct((B, K), f32),
        jax.ShapeDtypeStruct((B * K, K), f32),
    )
    scratch = (
        [pltpu.VMEM((BPW, NUM_REL), f32),
         pltpu.VMEM((BPW, K), i32),
         pltpu.VMEM((BPW, K), f32),
         pltpu.VMEM((H1C,), i32),
         pltpu.VMEM((H1C,), i32)]
        + [pltpu.VMEM((H1C, DIM), f32)] * 2
        + [pltpu.VMEM((H1C, K), i32)] * 4
        + [pltpu.VMEM((H1C, K), f32)] * 2
        + [pltpu.SemaphoreType.DMA] * 6
    )
    run = pl.kernel(_sc2_body, out_type=out_type,
                    mesh=plsc.VectorSubcoreMesh(**_MESH),
                    scratch_types=scratch, compiler_params=_SC_PARAMS)
    return run(e1_flat, r1, es, ent_emb, adj_ent, adj_rel)


H2R = H2_CHUNK // K


def _sc3_body(e2_hbm, u2_hbm, ent_hbm, ragg_out,
              idx0, idx1, buf0, buf1, u2c0, u2c1, ra0, ra1,
              sg0, sg1, sw0, sw1):
    w = _wid()
    npw = (B * K * K) // NW

    def reduce_chunk(buf_v, u2c_v, ra_v):
        @plsc.parallel_loop(0, H2R, unroll=4)
        def row(r):
            wts = u2c_v[r]
            prods = [wts[k] * buf_v[r * K + k] for k in range(K)]
            while len(prods) > 1:
                prods = [prods[i] + prods[i + 1]
                         for i in range(0, len(prods), 2)]
            ra_v[r // K, pl.ds((r % K) * DIM, DIM)] = prods[0]

    def pair(i2, _):
        o0 = w * npw + (2 * i2) * H2_CHUNK
        o1 = o0 + H2_CHUNK
        r0 = o0 // K
        r1 = o1 // K
        pltpu.sync_copy(e2_hbm.at[pl.ds(o0, H2_CHUNK)], idx0)
        g0 = pltpu.async_copy(ent_hbm.at[idx0], buf0, sg0)
        pltpu.sync_copy(e2_hbm.at[pl.ds(o1, H2_CHUNK)], idx1)
        g1 = pltpu.async_copy(ent_hbm.at[idx1], buf1, sg1)
        pltpu.sync_copy(u2_hbm.at[pl.ds(r0, H2R)], u2c0)
        pltpu.sync_copy(u2_hbm.at[pl.ds(r1, H2R)], u2c1)
        g0.wait()
        reduce_chunk(buf0, u2c0, ra0)
        w0 = pltpu.async_copy(ra0, ragg_out.at[pl.ds(r0 // K, H2R // K)], sw0)
        g1.wait()
        reduce_chunk(buf1, u2c1, ra1)
        w1 = pltpu.async_copy(ra1, ragg_out.at[pl.ds(r1 // K, H2R // K)], sw1)
        w0.wait()
        w1.wait()
        return 0

    lax.fori_loop(0, npw // (2 * H2_CHUNK), pair, 0)


def _sc3(e2_flat, u2, ent_emb):
    f32, i32 = jnp.float32, jnp.int32
    out_type = jax.ShapeDtypeStruct((B, K * DIM), f32)
    scratch = [
        pltpu.VMEM((H2_CHUNK,), i32),
        pltpu.VMEM((H2_CHUNK,), i32),
        pltpu.VMEM((H2_CHUNK, DIM), f32),
        pltpu.VMEM((H2_CHUNK, DIM), f32),
        pltpu.VMEM((H2R, K), f32),
        pltpu.VMEM((H2R, K), f32),
        pltpu.VMEM((H2R // K, K * DIM), f32),
        pltpu.VMEM((H2R // K, K * DIM), f32),
        pltpu.SemaphoreType.DMA,
        pltpu.SemaphoreType.DMA,
        pltpu.SemaphoreType.DMA,
        pltpu.SemaphoreType.DMA,
    ]
    run = pl.kernel(_sc3_body, out_type=out_type,
                    mesh=plsc.VectorSubcoreMesh(**_MESH),
                    scratch_types=scratch, compiler_params=_SC_PARAMS)
    return run(e2_flat, u2, ent_emb)


def _tca_body(ue_ref, relT_ref, es_ref):
    es_ref[...] = jnp.exp(
        jnp.dot(ue_ref[...], relT_ref[...], preferred_element_type=jnp.float32))


def _tca(ue, relT):
    bb = 2048
    return pl.pallas_call(
        _tca_body,
        grid=(B // bb,),
        in_specs=[
            pl.BlockSpec((bb, DIM), lambda i: (i, 0)),
            pl.BlockSpec((DIM, NUM_REL), lambda i: (0, 0)),
        ],
        out_specs=pl.BlockSpec((bb, NUM_REL), lambda i: (i, 0)),
        out_shape=jax.ShapeDtypeStruct((B, NUM_REL), jnp.float32),
    )(ue, relT)


def _sigmoid(x):
    return 1.0 / (1.0 + jnp.exp(-x))


BB = 1024


def _tcbf_body(ue_ref, v0_ref, u1_ref, v1p_ref, raggp_ref, u2p_ref,
               w_ref, b_ref, w256_ref, b256_ref, m_ref, rmat_ref, g_ref,
               out_ref):
    W = w_ref[...]
    bvec = b_ref[...]
    usum2p = jnp.dot(u2p_ref[...], m_ref[...],
                     preferred_element_type=jnp.float32)
    agg1p = raggp_ref[...] / usum2p
    v1p = v1p_ref[...]
    h1p = _sigmoid(jnp.dot(v1p + agg1p, w256_ref[...],
                           preferred_element_type=jnp.float32) + b256_ref[...])
    u1 = u1_ref[...]
    n1 = u1 / jnp.sum(u1, axis=1, keepdims=True)
    n1rep = jnp.dot(n1, rmat_ref[...],
                    preferred_element_type=jnp.float32)
    G = g_ref[...]
    agg0 = jnp.dot(n1rep * v1p, G, preferred_element_type=jnp.float32)
    h0 = _sigmoid(jnp.dot(v0_ref[...] + agg0, W,
                          preferred_element_type=jnp.float32) + bvec)
    agg0b = jnp.dot(n1rep * h1p, G, preferred_element_type=jnp.float32)
    t = jnp.dot(h0 + agg0b, W, preferred_element_type=jnp.float32) + bvec
    e2t = jnp.exp(-2.0 * t)
    item = (1.0 - e2t) / (1.0 + e2t)
    sc = jnp.sum(ue_ref[...] * item, axis=1, keepdims=True)
    out_ref[...] = _sigmoid(sc)


def _tcbf(ue, v0, u1, v1p, raggp, u2p, W, bvec, w256, b256, mmat, rmat, gmat):
    KD = K * DIM
    return pl.pallas_call(
        _tcbf_body,
        grid=(B // BB,),
        in_specs=[
            pl.BlockSpec((BB, DIM), lambda i: (i, 0)),
            pl.BlockSpec((BB, DIM), lambda i: (i, 0)),
            pl.BlockSpec((BB, K), lambda i: (i, 0)),
            pl.BlockSpec((BB, KD), lambda i: (i, 0)),
            pl.BlockSpec((BB, KD), lambda i: (i, 0)),
            pl.BlockSpec((BB, KD), lambda i: (i, 0)),
            pl.BlockSpec((DIM, DIM), lambda i: (0, 0)),
            pl.BlockSpec((1, DIM), lambda i: (0, 0)),
            pl.BlockSpec((KD, KD), lambda i: (0, 0)),
            pl.BlockSpec((1, KD), lambda i: (0, 0)),
            pl.BlockSpec((KD, KD), lambda i: (0, 0)),
            pl.BlockSpec((K, KD), lambda i: (0, 0)),
            pl.BlockSpec((KD, DIM), lambda i: (0, 0)),
        ],
        out_specs=pl.BlockSpec((BB, 1), lambda i: (i, 0)),
        out_shape=jax.ShapeDtypeStruct((B, 1), jnp.float32),
    )(ue, v0, u1, v1p, raggp, u2p, W, bvec, w256, b256, mmat, rmat, gmat)


def kernel(u, train_nids, adj_ent, adj_rel, usr_emb, rel_emb, ent_emb, W, b):
    u = u.astype(jnp.int32)
    train_nids = train_nids.astype(jnp.int32)
    adj_ent = adj_ent.astype(jnp.int32)
    adj_rel = adj_rel.astype(jnp.int32)

    ue = _sc1a(u, usr_emb)
    es = _tca(ue, rel_emb.T)
    v0, e1, r1 = _sc1b(train_nids, ent_emb, adj_ent, adj_rel)
    v1, e2, u1, u2 = _sc2(e1.reshape(B * K), r1, es,
                          ent_emb, adj_ent, adj_rel)
    raggp = _sc3(e2.reshape(B * K * K), u2, ent_emb)

    eyeK = np.eye(K, dtype=np.float32)
    rmat = jnp.asarray(np.kron(eyeK, np.ones((1, DIM), np.float32)))
    gmat = jnp.asarray(np.kron(np.ones((K, 1), np.float32), eyeK))
    mmat = jnp.asarray(np.kron(eyeK, np.ones((K, K * DIM // K), np.float32)))
    w256 = jnp.kron(jnp.eye(K, dtype=jnp.float32), W)
    b256 = jnp.tile(b, K).reshape(1, K * DIM)
    bvec = b.reshape(1, DIM)
    out = _tcbf(ue, v0, u1, v1.reshape(B, K * DIM), raggp,
                u2.reshape(B, K * DIM), W, bvec, w256, b256, mmat, rmat, gmat)
    return out.reshape(B)


# speedup vs baseline: 1.0352x; 1.0352x over previous
import jax
import jax.numpy as jnp
import numpy as np
from jax import lax
from jax.experimental import pallas as pl
from jax.experimental.pallas import tpu as pltpu
from jax.experimental.pallas import tpu_sc as plsc

NUM_ENT = 100000
NUM_REL = 32
DIM = 16
K = 16
B = 16384

NC = 2
NS = 16
NW = NC * NS
BPW = B // NW
H1_PER_W = BPW * K
H1_CHUNK = 1024
H2_CHUNK = 2048

_MESH = dict(core_axis_name="c", subcore_axis_name="s")
_SC_PARAMS = pltpu.CompilerParams(use_tc_tiling_on_sc=False,
                                  needs_layout_passes=False)


def _wid():
    return lax.axis_index("s") * NC + lax.axis_index("c")


def _sc1a_body(u_hbm, usr_hbm, ue_out, idx_v, ue_v, sem):
    sl = pl.ds(_wid() * BPW, BPW)
    pltpu.sync_copy(u_hbm.at[sl], idx_v)
    pltpu.async_copy(usr_hbm.at[idx_v], ue_v, sem).wait()
    pltpu.sync_copy(ue_v, ue_out.at[sl])


def _sc1a(u, usr_emb):
    run = pl.kernel(
        _sc1a_body,
        out_type=jax.ShapeDtypeStruct((B, DIM), jnp.float32),
        mesh=plsc.VectorSubcoreMesh(**_MESH),
        scratch_types=[
            pltpu.VMEM((BPW,), jnp.int32),
            pltpu.VMEM((BPW, DIM), jnp.float32),
            pltpu.SemaphoreType.DMA,
        ],
        compiler_params=_SC_PARAMS)
    return run(u, usr_emb)


def _sc1b_body(tn_hbm, ent_hbm, adje_hbm, adjr_hbm,
               v0_out, e1_out, r1_out,
               idx_v, v0_v, e1_v, r1_v, sem, sem2, sem3):
    sl = pl.ds(_wid() * BPW, BPW)
    pltpu.sync_copy(tn_hbm.at[sl], idx_v)
    g1 = pltpu.async_copy(ent_hbm.at[idx_v], v0_v, sem)
    g2 = pltpu.async_copy(adje_hbm.at[idx_v], e1_v, sem2)
    g3 = pltpu.async_copy(adjr_hbm.at[idx_v], r1_v, sem3)
    g1.wait()
    pltpu.sync_copy(v0_v, v0_out.at[sl])
    g2.wait()
    pltpu.sync_copy(e1_v, e1_out.at[sl])
    g3.wait()
    pltpu.sync_copy(r1_v, r1_out.at[sl])


def _sc1b(tn, ent_emb, adj_ent, adj_rel):
    f32, i32 = jnp.float32, jnp.int32
    out_type = (
        jax.ShapeDtypeStruct((B, DIM), f32),
        jax.ShapeDtypeStruct((B, K), i32),
        jax.ShapeDtypeStruct((B, K), i32),
    )
    scratch = [
        pltpu.VMEM((BPW,), i32),
        pltpu.VMEM((BPW, DIM), f32),
        pltpu.VMEM((BPW, K), i32),
        pltpu.VMEM((BPW, K), i32),
        pltpu.SemaphoreType.DMA,
        pltpu.SemaphoreType.DMA,
        pltpu.SemaphoreType.DMA,
    ]
    run = pl.kernel(_sc1b_body, out_type=out_type,
                    mesh=plsc.VectorSubcoreMesh(**_MESH),
                    scratch_types=scratch, compiler_params=_SC_PARAMS)
    return run(tn, ent_emb, adj_ent, adj_rel)


H1C = 512


def _sc2_body(e1_hbm, r1_hbm, es_hbm, ent_hbm, adje_hbm, adjr_hbm,
              v1_out, e2_out, u1_out, u2_out,
              es_v, r1_v, u1_v,
              idx0, idx1, v1a, v1b, e2a, e2b, r2a, r2b, u2a, u2b,
              s1a, s2a, s3a, s1b, s2b, s3b):
    w = _wid()
    b0 = w * BPW
    pltpu.sync_copy(es_hbm.at[pl.ds(b0, BPW)], es_v)
    pltpu.sync_copy(r1_hbm.at[pl.ds(b0, BPW)], r1_v)

    @plsc.parallel_loop(0, BPW, unroll=4)
    def u1_row(r):
        rows = jnp.full((16,), r, jnp.int32)
        u1_v[r] = plsc.load_gather(es_v, [rows, r1_v[r]])
    pltpu.sync_copy(u1_v, u1_out.at[pl.ds(b0, BPW)])

    def u2_loop(base_row, r2_v, u2_v):
        @plsc.parallel_loop(0, H1C, unroll=4)
        def u2_row(r):
            b_local = (base_row + r) // K
            rows = jnp.full((16,), b_local, jnp.int32)
            u2_v[r] = plsc.load_gather(es_v, [rows, r2_v[r]])

    def pair(i2, _):
        base0 = 2 * i2 * H1C
        base1 = base0 + H1C
        o0 = w * H1_PER_W + base0
        sl0 = pl.ds(o0, H1C)
        sl1 = pl.ds(o0 + H1C, H1C)
        pltpu.sync_copy(e1_hbm.at[sl0], idx0)
        g1a = pltpu.async_copy(ent_hbm.at[idx0], v1a, s1a)
        g2a = pltpu.async_copy(adje_hbm.at[idx0], e2a, s2a)
        g3a = pltpu.async_copy(adjr_hbm.at[idx0], r2a, s3a)
        pltpu.sync_copy(e1_hbm.at[sl1], idx1)
        g1b = pltpu.async_copy(ent_hbm.at[idx1], v1b, s1b)
        g2b = pltpu.async_copy(adje_hbm.at[idx1], e2b, s2b)
        g3b = pltpu.async_copy(adjr_hbm.at[idx1], r2b, s3b)
        g1a.wait()
        wa1 = pltpu.async_copy(v1a, v1_out.at[sl0], s1a)
        g2a.wait()
        wa2 = pltpu.async_copy(e2a, e2_out.at[sl0], s2a)
        g3a.wait()
        u2_loop(base0, r2a, u2a)
        wa3 = pltpu.async_copy(u2a, u2_out.at[sl0], s3a)
        g1b.wait()
        wb1 = pltpu.async_copy(v1b, v1_out.at[sl1], s1b)
        g2b.wait()
        wb2 = pltpu.async_copy(e2b, e2_out.at[sl1], s2b)
        g3b.wait()
        u2_loop(base1, r2b, u2b)
        wb3 = pltpu.async_copy(u2b, u2_out.at[sl1], s3b)
        wa1.wait()
        wa2.wait()
        wa3.wait()
        wb1.wait()
        wb2.wait()
        wb3.wait()
        return 0

    lax.fori_loop(0, H1_PER_W // (2 * H1C), pair, 0)


def _sc2(e1_flat, r1, es, ent_emb, adj_ent, adj_rel):
    f32, i32 = jnp.float32, jnp.int32
    out_type = (
        jax.ShapeDtypeStruct((B * K, DIM), f32),
        jax.ShapeDtypeStruct((B * K, K), i32),
        jax.ShapeDtypeStruct((B, K), f32),
        jax.ShapeDtypeStruct((B * K, K), f32),
    )
    scratch = (
        [pltpu.VMEM((BPW, NUM_REL), f32),
         pltpu.VMEM((BPW, K), i32),
         pltpu.VMEM((BPW, K), f32),
         pltpu.VMEM((H1C,), i32),
         pltpu.VMEM((H1C,), i32)]
        + [pltpu.VMEM((H1C, DIM), f32)] * 2
        + [pltpu.VMEM((H1C, K), i32)] * 4
        + [pltpu.VMEM((H1C, K), f32)] * 2
        + [pltpu.SemaphoreType.DMA] * 6
    )
    run = pl.kernel(_sc2_body, out_type=out_type,
                    mesh=plsc.VectorSubcoreMesh(**_MESH),
                    scratch_types=scratch, compiler_params=_SC_PARAMS)
    return run(e1_flat, r1, es, ent_emb, adj_ent, adj_rel)


H2R = H2_CHUNK // K


def _sc3_body(e2_hbm, u2_hbm, ent_hbm, ragg_out,
              idx0, idx1, buf0, buf1, u2c0, u2c1, ra0, ra1,
              sg0, sg1, sw0, sw1):
    w = _wid()
    npw = (B * K * K) // NW

    def reduce_chunk(buf_v, u2c_v, ra_v):
        @plsc.parallel_loop(0, H2R, unroll=2)
        def row(r):
            wts = u2c_v[r]
            prods = [wts[k] * buf_v[r * K + k] for k in range(K)]
            while len(prods) > 1:
                prods = [prods[i] + prods[i + 1]
                         for i in range(0, len(prods), 2)]
            ra_v[r // K, pl.ds((r % K) * DIM, DIM)] = prods[0]

    def pair(i2, _):
        o0 = w * npw + (2 * i2) * H2_CHUNK
        o1 = o0 + H2_CHUNK
        r0 = o0 // K
        r1 = o1 // K
        pltpu.sync_copy(e2_hbm.at[pl.ds(o0, H2_CHUNK)], idx0)
        g0 = pltpu.async_copy(ent_hbm.at[idx0], buf0, sg0)
        pltpu.sync_copy(e2_hbm.at[pl.ds(o1, H2_CHUNK)], idx1)
        g1 = pltpu.async_copy(ent_hbm.at[idx1], buf1, sg1)
        pltpu.sync_copy(u2_hbm.at[pl.ds(r0, H2R)], u2c0)
        pltpu.sync_copy(u2_hbm.at[pl.ds(r1, H2R)], u2c1)
        g0.wait()
        reduce_chunk(buf0, u2c0, ra0)
        w0 = pltpu.async_copy(ra0, ragg_out.at[pl.ds(r0 // K, H2R // K)], sw0)
        g1.wait()
        reduce_chunk(buf1, u2c1, ra1)
        w1 = pltpu.async_copy(ra1, ragg_out.at[pl.ds(r1 // K, H2R // K)], sw1)
        w0.wait()
        w1.wait()
        return 0

    lax.fori_loop(0, npw // (2 * H2_CHUNK), pair, 0)


def _sc3(e2_flat, u2, ent_emb):
    f32, i32 = jnp.float32, jnp.int32
    out_type = jax.ShapeDtypeStruct((B, K * DIM), f32)
    scratch = [
        pltpu.VMEM((H2_CHUNK,), i32),
        pltpu.VMEM((H2_CHUNK,), i32),
        pltpu.VMEM((H2_CHUNK, DIM), f32),
        pltpu.VMEM((H2_CHUNK, DIM), f32),
        pltpu.VMEM((H2R, K), f32),
        pltpu.VMEM((H2R, K), f32),
        pltpu.VMEM((H2R // K, K * DIM), f32),
        pltpu.VMEM((H2R // K, K * DIM), f32),
        pltpu.SemaphoreType.DMA,
        pltpu.SemaphoreType.DMA,
        pltpu.SemaphoreType.DMA,
        pltpu.SemaphoreType.DMA,
    ]
    run = pl.kernel(_sc3_body, out_type=out_type,
                    mesh=plsc.VectorSubcoreMesh(**_MESH),
                    scratch_types=scratch, compiler_params=_SC_PARAMS)
    return run(e2_flat, u2, ent_emb)


def _tca_body(ue_ref, relT_ref, es_ref):
    es_ref[...] = jnp.exp(
        jnp.dot(ue_ref[...], relT_ref[...], preferred_element_type=jnp.float32))


def _tca(ue, relT):
    bb = 2048
    return pl.pallas_call(
        _tca_body,
        grid=(B // bb,),
        in_specs=[
            pl.BlockSpec((bb, DIM), lambda i: (i, 0)),
            pl.BlockSpec((DIM, NUM_REL), lambda i: (0, 0)),
        ],
        out_specs=pl.BlockSpec((bb, NUM_REL), lambda i: (i, 0)),
        out_shape=jax.ShapeDtypeStruct((B, NUM_REL), jnp.float32),
    )(ue, relT)


def _sigmoid(x):
    return 1.0 / (1.0 + jnp.exp(-x))


BB = 1024


def _tcbf_body(ue_ref, v0_ref, u1_ref, v1p_ref, raggp_ref, u2p_ref,
               w_ref, b_ref, w256_ref, b256_ref, m_ref, rmat_ref, g_ref,
               out_ref):
    W = w_ref[...]
    bvec = b_ref[...]
    usum2p = jnp.dot(u2p_ref[...], m_ref[...],
                     preferred_element_type=jnp.float32)
    agg1p = raggp_ref[...] / usum2p
    v1p = v1p_ref[...]
    h1p = _sigmoid(jnp.dot(v1p + agg1p, w256_ref[...],
                           preferred_element_type=jnp.float32) + b256_ref[...])
    u1 = u1_ref[...]
    n1 = u1 / jnp.sum(u1, axis=1, keepdims=True)
    n1rep = jnp.dot(n1, rmat_ref[...],
                    preferred_element_type=jnp.float32)
    G = g_ref[...]
    agg0 = jnp.dot(n1rep * v1p, G, preferred_element_type=jnp.float32)
    h0 = _sigmoid(jnp.dot(v0_ref[...] + agg0, W,
                          preferred_element_type=jnp.float32) + bvec)
    agg0b = jnp.dot(n1rep * h1p, G, preferred_element_type=jnp.float32)
    t = jnp.dot(h0 + agg0b, W, preferred_element_type=jnp.float32) + bvec
    e2t = jnp.exp(-2.0 * t)
    item = (1.0 - e2t) / (1.0 + e2t)
    sc = jnp.sum(ue_ref[...] * item, axis=1, keepdims=True)
    out_ref[...] = _sigmoid(sc)


def _tcbf(ue, v0, u1, v1p, raggp, u2p, W, bvec, w256, b256, mmat, rmat, gmat):
    KD = K * DIM
    return pl.pallas_call(
        _tcbf_body,
        grid=(B // BB,),
        in_specs=[
            pl.BlockSpec((BB, DIM), lambda i: (i, 0)),
            pl.BlockSpec((BB, DIM), lambda i: (i, 0)),
            pl.BlockSpec((BB, K), lambda i: (i, 0)),
            pl.BlockSpec((BB, KD), lambda i: (i, 0)),
            pl.BlockSpec((BB, KD), lambda i: (i, 0)),
            pl.BlockSpec((BB, KD), lambda i: (i, 0)),
            pl.BlockSpec((DIM, DIM), lambda i: (0, 0)),
            pl.BlockSpec((1, DIM), lambda i: (0, 0)),
            pl.BlockSpec((KD, KD), lambda i: (0, 0)),
            pl.BlockSpec((1, KD), lambda i: (0, 0)),
            pl.BlockSpec((KD, KD), lambda i: (0, 0)),
            pl.BlockSpec((K, KD), lambda i: (0, 0)),
            pl.BlockSpec((KD, DIM), lambda i: (0, 0)),
        ],
        out_specs=pl.BlockSpec((BB, 1), lambda i: (i, 0)),
        out_shape=jax.ShapeDtypeStruct((B, 1), jnp.float32),
    )(ue, v0, u1, v1p, raggp, u2p, W, bvec, w256, b256, mmat, rmat, gmat)


def kernel(u, train_nids, adj_ent, adj_rel, usr_emb, rel_emb, ent_emb, W, b):
    u = u.astype(jnp.int32)
    train_nids = train_nids.astype(jnp.int32)
    adj_ent = adj_ent.astype(jnp.int32)
    adj_rel = adj_rel.astype(jnp.int32)

    ue = _sc1a(u, usr_emb)
    es = _tca(ue, rel_emb.T)
    v0, e1, r1 = _sc1b(train_nids, ent_emb, adj_ent, adj_rel)
    v1, e2, u1, u2 = _sc2(e1.reshape(B * K), r1, es,
                          ent_emb, adj_ent, adj_rel)
    raggp = _sc3(e2.reshape(B * K * K), u2, ent_emb)

    eyeK = np.eye(K, dtype=np.float32)
    rmat = jnp.asarray(np.kron(eyeK, np.ones((1, DIM), np.float32)))
    gmat = jnp.asarray(np.kron(np.ones((K, 1), np.float32), eyeK))
    mmat = jnp.asarray(np.kron(eyeK, np.ones((K, K * DIM // K), np.float32)))
    w256 = jnp.kron(jnp.eye(K, dtype=jnp.float32), W)
    b256 = jnp.tile(b, K).reshape(1, K * DIM)
    bvec = b.reshape(1, DIM)
    out = _tcbf(ue, v0, u1, v1.reshape(B, K * DIM), raggp,
                u2.reshape(B, K * DIM), W, bvec, w256, b256, mmat, rmat, gmat)
    return out.reshape(B)
